# baseline (speedup 1.0000x reference)
import jax
import jax.numpy as jnp
from jax.experimental import pallas as pl

_ALPHA = 0.1
_EPS2 = 1e-24

_BATCH = 4096
_BIT = 64
_GRID = 4
_ROWS = _BATCH // 2 // _GRID


def _loss_kernel(u_ref, img_ref, out_ref):
    u = u_ref[...]
    img = img_ref[...]
    i = jax.lax.broadcasted_iota(jnp.int32, (128, 128), 0)
    j = jax.lax.broadcasted_iota(jnp.int32, (128, 128), 1)
    m = jnp.where((i < _BIT) == (j < _BIT), 1.0, 0.0)
    s = jnp.where(img > 0.0, 1.0, jnp.where(img < 0.0, -1.0, 0.0))
    k = jnp.dot(s * s, m, preferred_element_type=jnp.float32)
    nsq = jnp.dot(u * u, m, preferred_element_type=jnp.float32)
    a = jax.lax.rsqrt(jnp.maximum(k, _EPS2))
    b = jax.lax.rsqrt(jnp.maximum(nsq, _EPS2))
    diff = s * a - u * b
    d2 = diff * diff
    factor = jnp.where(s * u < 0.0, 2.0, 1.0)
    contrib = d2 * factor + _ALPHA * jnp.abs(diff)
    partial = jnp.reshape(jnp.sum(contrib) * (1.0 / _BATCH), (1, 1))

    @pl.when(pl.program_id(0) == 0)
    def _init():
        out_ref[...] = jnp.zeros_like(out_ref)

    out_ref[...] += partial


def kernel(u, y, ind, image, U, sign_L):
    u2 = jnp.reshape(u, (_BATCH // 2, 2 * _BIT))
    img2 = jnp.reshape(image, (_BATCH // 2, 2 * _BIT))
    out = pl.pallas_call(
        _loss_kernel,
        grid=(_GRID,),
        in_specs=[
            pl.BlockSpec((_ROWS, 2 * _BIT), lambda i: (i, 0)),
            pl.BlockSpec((_ROWS, 2 * _BIT), lambda i: (i, 0)),
        ],
        out_specs=pl.BlockSpec((1, 1), lambda i: (0, 0)),
        out_shape=jax.ShapeDtypeStruct((1, 1), jnp.float32),
    )(u2, img2)
    return jnp.reshape(out, ())


# speedup vs baseline: 1.1244x; 1.1244x over previous
import jax
import jax.numpy as jnp
from jax.experimental import pallas as pl
from jax.experimental.pallas import tpu as pltpu

_ALPHA = 0.1
_EPS2 = 1e-24

_BATCH = 4096
_BIT = 64
_R = _BATCH // 2
_H = _R // 2


def _chunk_loss(u, img, m):
    s = jnp.where(img > 0.0, 1.0, jnp.where(img < 0.0, -1.0, 0.0))
    k = jnp.dot(s * s, m, preferred_element_type=jnp.float32)
    nsq = jnp.dot(u * u, m, preferred_element_type=jnp.float32)
    a = jax.lax.rsqrt(jnp.maximum(k, _EPS2))
    b = jax.lax.rsqrt(jnp.maximum(nsq, _EPS2))
    diff = s * a - u * b
    d2 = diff * diff
    factor = jnp.where(s * u < 0.0, 2.0, 1.0)
    contrib = d2 * factor + _ALPHA * jnp.abs(diff)
    return jnp.sum(contrib)


def _loss_kernel(u_hbm, img_hbm, out_ref, u_vmem, img_vmem, s0, s1, s2, s3):
    cu0 = pltpu.make_async_copy(u_hbm.at[pl.ds(0, _H)], u_vmem.at[pl.ds(0, _H)], s0)
    ci0 = pltpu.make_async_copy(img_hbm.at[pl.ds(0, _H)], img_vmem.at[pl.ds(0, _H)], s1)
    cu1 = pltpu.make_async_copy(u_hbm.at[pl.ds(_H, _H)], u_vmem.at[pl.ds(_H, _H)], s2)
    ci1 = pltpu.make_async_copy(img_hbm.at[pl.ds(_H, _H)], img_vmem.at[pl.ds(_H, _H)], s3)
    cu0.start()
    ci0.start()
    cu1.start()
    ci1.start()
    i = jax.lax.broadcasted_iota(jnp.int32, (128, 128), 0)
    j = jax.lax.broadcasted_iota(jnp.int32, (128, 128), 1)
    m = jnp.where((i < _BIT) == (j < _BIT), 1.0, 0.0)
    cu0.wait()
    ci0.wait()
    p0 = _chunk_loss(u_vmem[0:_H], img_vmem[0:_H], m)
    cu1.wait()
    ci1.wait()
    p1 = _chunk_loss(u_vmem[_H : 2 * _H], img_vmem[_H : 2 * _H], m)
    out_ref[...] = jnp.reshape((p0 + p1) * (1.0 / _BATCH), (1, 1))


def kernel(u, y, ind, image, U, sign_L):
    u2 = jnp.reshape(u, (_R, 2 * _BIT))
    img2 = jnp.reshape(image, (_R, 2 * _BIT))
    out = pl.pallas_call(
        _loss_kernel,
        in_specs=[
            pl.BlockSpec(memory_space=pl.ANY),
            pl.BlockSpec(memory_space=pl.ANY),
        ],
        out_specs=pl.BlockSpec(memory_space=pltpu.MemorySpace.VMEM),
        out_shape=jax.ShapeDtypeStruct((1, 1), jnp.float32),
        scratch_shapes=[
            pltpu.VMEM((_R, 2 * _BIT), jnp.float32),
            pltpu.VMEM((_R, 2 * _BIT), jnp.float32),
            pltpu.SemaphoreType.DMA,
            pltpu.SemaphoreType.DMA,
            pltpu.SemaphoreType.DMA,
            pltpu.SemaphoreType.DMA,
        ],
    )(u2, img2)
    return jnp.reshape(out, ())
